# Initial kernel scaffold; baseline (speedup 1.0000x reference)
import jax
import jax.numpy as jnp
from jax.experimental import pallas as pl


def kernel(inputs, evaluate_table, takecare_table, vector_table, idx_table):
    raise NotImplementedError("write your pallas kernel here")


# speedup vs baseline: 289.7379x; 289.7379x over previous
import functools

import jax
import jax.numpy as jnp
from jax import lax
from jax.experimental import pallas as pl
from jax.experimental.pallas import tpu as pltpu
from jax.experimental.pallas import tpu_sc as plsc

L = 16


@functools.lru_cache(maxsize=None)
def _build(B, T, E):
    W = 2 * E + 1
    info = plsc.get_sparse_core_info()
    NC, NS = info.num_cores, info.num_subcores
    NW = NC * NS
    BPW = B // NW
    GROUPS = BPW // L
    mesh = plsc.VectorSubcoreMesh(core_axis_name="c", subcore_axis_name="s")

    @functools.partial(
        pl.kernel,
        mesh=mesh,
        out_type=jax.ShapeDtypeStruct((B,), jnp.float32),
        compiler_params=pltpu.CompilerParams(needs_layout_passes=False),
        scratch_types=[
            pltpu.VMEM((T,), jnp.float32),
            pltpu.VMEM((T,), jnp.float32),
            pltpu.VMEM((T,), jnp.float32),
            pltpu.VMEM((BPW,), jnp.float32),
            pltpu.VMEM((BPW,), jnp.float32),
        ],
    )
    def hwnet_sc(x_hbm, ev_hbm, tc_hbm, vec_hbm, out_hbm,
                 ev_v, tc_v, vec_v, x_v, o_v):
        wid = lax.axis_index("s") * NC + lax.axis_index("c")
        base = wid * BPW
        pltpu.sync_copy(ev_hbm, ev_v)
        pltpu.sync_copy(tc_hbm, tc_v)
        pltpu.sync_copy(vec_hbm, vec_v)
        pltpu.sync_copy(x_hbm.at[pl.ds(base, BPW)], x_v)

        def lower_bound(key):
            pos = jnp.zeros((L,), jnp.int32)
            half = T // 2
            while half >= 1:
                probe = plsc.load_gather(ev_v, [pos + (half - 1)])
                pos = jnp.where(probe < key, pos + half, pos)
                half //= 2
            return pos

        def group(j, _):
            x = x_v[pl.ds(j * L, L)]
            pos = lower_bound(x)
            a = jnp.maximum(pos - 1, 0)
            ea = plsc.load_gather(ev_v, [a])
            eb = plsc.load_gather(ev_v, [pos])
            ra = x - ea
            rb = x - eb
            da = ra * ra
            db = rb * rb
            vstar = jnp.where(da <= db, ea, eb)
            idx = lower_bound(vstar)
            tc = plsc.load_gather(tc_v, [idx])
            dmin = jnp.minimum(da, db)
            start = jnp.clip(idx, E, T - E - 1) - E

            def wstep(w, carry):
                num, den = carry
                ew = plsc.load_gather(ev_v, [start + w])
                vw = plsc.load_gather(vec_v, [start + w])
                r = x - ew
                d = r * r
                e = jnp.exp(tc * (dmin - d))
                return (num + vw * e, den + e)

            zero = jnp.zeros((L,), jnp.float32)
            num, den = lax.fori_loop(0, W, wstep, (zero, zero))
            o_v[pl.ds(j * L, L)] = num / den
            return 0

        lax.fori_loop(0, GROUPS, group, 0)
        pltpu.sync_copy(o_v, out_hbm.at[pl.ds(base, BPW)])

    return hwnet_sc


def kernel(inputs, evaluate_table, takecare_table, vector_table, idx_table):
    B = inputs.shape[0]
    T = evaluate_table.shape[0]
    E = (idx_table.shape[0] - 1) // 2
    D = vector_table.shape[1]
    assert D == 1
    fn = _build(B, T, E)
    out = fn(
        inputs.reshape(B),
        evaluate_table.reshape(T),
        takecare_table.reshape(T),
        vector_table.reshape(T),
    )
    return out.reshape(B, D)


# speedup vs baseline: 317.4903x; 1.0958x over previous
import functools

import jax
import jax.numpy as jnp
from jax import lax
from jax.experimental import pallas as pl
from jax.experimental.pallas import tpu as pltpu
from jax.experimental.pallas import tpu_sc as plsc

L = 16


@functools.lru_cache(maxsize=None)
def _build(B, T, E):
    W = 2 * E + 1
    info = plsc.get_sparse_core_info()
    NC, NS = info.num_cores, info.num_subcores
    NW = NC * NS
    BPW = B // NW
    GROUPS = BPW // L
    mesh = plsc.VectorSubcoreMesh(core_axis_name="c", subcore_axis_name="s")

    @functools.partial(
        pl.kernel,
        mesh=mesh,
        out_type=jax.ShapeDtypeStruct((B,), jnp.float32),
        compiler_params=pltpu.CompilerParams(needs_layout_passes=False),
        scratch_types=[
            pltpu.VMEM((T,), jnp.float32),
            pltpu.VMEM((T,), jnp.float32),
            pltpu.VMEM((T,), jnp.float32),
            pltpu.VMEM((BPW,), jnp.float32),
            pltpu.VMEM((BPW,), jnp.float32),
        ],
    )
    def hwnet_sc(x_hbm, ev_hbm, tc_hbm, vec_hbm, out_hbm,
                 ev_v, tc_v, vec_v, x_v, o_v):
        wid = lax.axis_index("s") * NC + lax.axis_index("c")
        base = wid * BPW
        pltpu.sync_copy(ev_hbm, ev_v)
        pltpu.sync_copy(tc_hbm, tc_v)
        pltpu.sync_copy(vec_hbm, vec_v)
        pltpu.sync_copy(x_hbm.at[pl.ds(base, BPW)], x_v)

        def lower_bound2(k0, k1):
            p0 = jnp.zeros((L,), jnp.int32)
            p1 = jnp.zeros((L,), jnp.int32)
            half = T // 2
            while half >= 1:
                v0 = plsc.load_gather(ev_v, [p0 + (half - 1)])
                v1 = plsc.load_gather(ev_v, [p1 + (half - 1)])
                p0 = jnp.where(v0 < k0, p0 + half, p0)
                p1 = jnp.where(v1 < k1, p1 + half, p1)
                half //= 2
            return p0, p1

        def nearest(x, pos):
            a = jnp.maximum(pos - 1, 0)
            ea = plsc.load_gather(ev_v, [a])
            eb = plsc.load_gather(ev_v, [pos])
            ra = x - ea
            rb = x - eb
            da = ra * ra
            db = rb * rb
            take_a = da <= db
            vstar = jnp.where(take_a, ea, eb)
            dmin = jnp.minimum(da, db)
            cand = jnp.where(take_a, a, pos)
            return vstar, dmin, cand

        def group_pair(jj, _):
            j0 = jj * 2
            x0 = x_v[pl.ds(j0 * L, L)]
            x1 = x_v[pl.ds(j0 * L + L, L)]
            pos0, pos1 = lower_bound2(x0, x1)
            v0, dmin0, c0 = nearest(x0, pos0)
            v1, dmin1, c1 = nearest(x1, pos1)
            pa0 = plsc.load_gather(ev_v, [jnp.maximum(c0 - 1, 0)])
            pa1 = plsc.load_gather(ev_v, [jnp.maximum(c1 - 1, 0)])
            dup0 = (pa0 == v0) & (c0 > 0)
            dup1 = (pa1 == v1) & (c1 > 0)
            any_dup = jnp.any(dup0) | jnp.any(dup1)

            def slow(_):
                f0, f1 = lower_bound2(v0, v1)
                return (jnp.where(dup0, f0, c0), jnp.where(dup1, f1, c1))

            idx0, idx1 = lax.cond(any_dup, slow, lambda _: (c0, c1), 0)

            tc0 = plsc.load_gather(tc_v, [idx0])
            tc1 = plsc.load_gather(tc_v, [idx1])
            s0 = jnp.clip(idx0, E, T - E - 1) - E
            s1 = jnp.clip(idx1, E, T - E - 1) - E

            def wstep(w, carry):
                n0, d0, n1, d1 = carry
                i0 = s0 + w
                i1 = s1 + w
                ew0 = plsc.load_gather(ev_v, [i0])
                vw0 = plsc.load_gather(vec_v, [i0])
                ew1 = plsc.load_gather(ev_v, [i1])
                vw1 = plsc.load_gather(vec_v, [i1])
                r0 = x0 - ew0
                r1 = x1 - ew1
                e0 = jnp.exp(tc0 * (dmin0 - r0 * r0))
                e1 = jnp.exp(tc1 * (dmin1 - r1 * r1))
                return (n0 + vw0 * e0, d0 + e0, n1 + vw1 * e1, d1 + e1)

            zero = jnp.zeros((L,), jnp.float32)
            n0, d0, n1, d1 = lax.fori_loop(
                0, W, wstep, (zero, zero, zero, zero), unroll=8)
            o_v[pl.ds(j0 * L, L)] = n0 / d0
            o_v[pl.ds(j0 * L + L, L)] = n1 / d1
            return 0

        lax.fori_loop(0, GROUPS // 2, group_pair, 0)
        pltpu.sync_copy(o_v, out_hbm.at[pl.ds(base, BPW)])

    return hwnet_sc


def kernel(inputs, evaluate_table, takecare_table, vector_table, idx_table):
    B = inputs.shape[0]
    T = evaluate_table.shape[0]
    E = (idx_table.shape[0] - 1) // 2
    D = vector_table.shape[1]
    assert D == 1
    fn = _build(B, T, E)
    out = fn(
        inputs.reshape(B),
        evaluate_table.reshape(T),
        takecare_table.reshape(T),
        vector_table.reshape(T),
    )
    return out.reshape(B, D)


# speedup vs baseline: 350.3155x; 1.1034x over previous
import functools

import jax
import jax.numpy as jnp
from jax import lax
from jax.experimental import pallas as pl
from jax.experimental.pallas import tpu as pltpu
from jax.experimental.pallas import tpu_sc as plsc

L = 16


@functools.lru_cache(maxsize=None)
def _build(B, T, E):
    W = 2 * E + 1
    info = plsc.get_sparse_core_info()
    NC, NS = info.num_cores, info.num_subcores
    NW = NC * NS
    BPW = B // NW
    GROUPS = BPW // L
    mesh = plsc.VectorSubcoreMesh(core_axis_name="c", subcore_axis_name="s")

    @functools.partial(
        pl.kernel,
        mesh=mesh,
        out_type=jax.ShapeDtypeStruct((B,), jnp.float32),
        compiler_params=pltpu.CompilerParams(needs_layout_passes=False),
        scratch_types=[
            pltpu.VMEM((T,), jnp.float32),
            pltpu.VMEM((T,), jnp.float32),
            pltpu.VMEM((BPW,), jnp.float32),
            pltpu.VMEM((BPW,), jnp.float32),
            pltpu.VMEM((BPW,), jnp.int32),
            pltpu.VMEM((BPW,), jnp.int32),
            pltpu.VMEM((BPW,), jnp.float32),
            pltpu.VMEM((BPW,), jnp.float32),
            pltpu.SemaphoreType.DMA,
            pltpu.SemaphoreType.DMA,
            pltpu.SemaphoreType.DMA,
        ],
    )
    def hwnet_sc(x_hbm, ev_hbm, tc_hbm, vec_hbm, out_hbm,
                 ev_v, vec_v, x_v, o_v, idx_v, st_v, dm_v, tcs_v,
                 sem_ev, sem_vec, sem_x):
        wid = lax.axis_index("s") * NC + lax.axis_index("c")
        base = wid * BPW
        h_ev = pltpu.async_copy(ev_hbm, ev_v, sem_ev)
        h_vec = pltpu.async_copy(vec_hbm, vec_v, sem_vec)
        h_x = pltpu.async_copy(x_hbm.at[pl.ds(base, BPW)], x_v, sem_x)
        h_ev.wait()
        h_x.wait()

        def lower_bound2(k0, k1):
            p0 = jnp.zeros((L,), jnp.int32)
            p1 = jnp.zeros((L,), jnp.int32)
            half = T // 2
            while half >= 1:
                v0 = plsc.load_gather(ev_v, [p0 + (half - 1)])
                v1 = plsc.load_gather(ev_v, [p1 + (half - 1)])
                p0 = jnp.where(v0 < k0, p0 + half, p0)
                p1 = jnp.where(v1 < k1, p1 + half, p1)
                half //= 2
            return p0, p1

        def nearest(x, pos):
            a = jnp.maximum(pos - 1, 0)
            ea = plsc.load_gather(ev_v, [a])
            eb = plsc.load_gather(ev_v, [pos])
            ra = x - ea
            rb = x - eb
            da = ra * ra
            db = rb * rb
            take_a = da <= db
            vstar = jnp.where(take_a, ea, eb)
            dmin = jnp.minimum(da, db)
            cand = jnp.where(take_a, a, pos)
            return vstar, dmin, cand

        def search_pair(jj, _):
            j0 = jj * 2
            x0 = x_v[pl.ds(j0 * L, L)]
            x1 = x_v[pl.ds(j0 * L + L, L)]
            pos0, pos1 = lower_bound2(x0, x1)
            v0, dmin0, c0 = nearest(x0, pos0)
            v1, dmin1, c1 = nearest(x1, pos1)
            pa0 = plsc.load_gather(ev_v, [jnp.maximum(c0 - 1, 0)])
            pa1 = plsc.load_gather(ev_v, [jnp.maximum(c1 - 1, 0)])
            dup0 = (pa0 == v0) & (c0 > 0)
            dup1 = (pa1 == v1) & (c1 > 0)
            any_dup = jnp.any(dup0) | jnp.any(dup1)

            def slow(_):
                f0, f1 = lower_bound2(v0, v1)
                return (jnp.where(dup0, f0, c0), jnp.where(dup1, f1, c1))

            idx0, idx1 = lax.cond(any_dup, slow, lambda _: (c0, c1), 0)

            idx_v[pl.ds(j0 * L, L)] = idx0
            idx_v[pl.ds(j0 * L + L, L)] = idx1
            st_v[pl.ds(j0 * L, L)] = jnp.clip(idx0, E, T - E - 1) - E
            st_v[pl.ds(j0 * L + L, L)] = jnp.clip(idx1, E, T - E - 1) - E
            dm_v[pl.ds(j0 * L, L)] = dmin0
            dm_v[pl.ds(j0 * L + L, L)] = dmin1
            return 0

        lax.fori_loop(0, GROUPS // 2, search_pair, 0)

        pltpu.async_copy(tc_hbm.at[idx_v], tcs_v, sem_ev).wait()
        h_vec.wait()

        def window_pair(jj, _):
            j0 = jj * 2
            o0 = j0 * L
            o1 = j0 * L + L
            x0 = x_v[pl.ds(o0, L)]
            x1 = x_v[pl.ds(o1, L)]
            s0 = st_v[pl.ds(o0, L)]
            s1 = st_v[pl.ds(o1, L)]
            dmin0 = dm_v[pl.ds(o0, L)]
            dmin1 = dm_v[pl.ds(o1, L)]
            tc0 = tcs_v[pl.ds(o0, L)]
            tc1 = tcs_v[pl.ds(o1, L)]

            def wstep(w, carry):
                n0, d0, n1, d1 = carry
                i0 = s0 + w
                i1 = s1 + w
                ew0 = plsc.load_gather(ev_v, [i0])
                vw0 = plsc.load_gather(vec_v, [i0])
                ew1 = plsc.load_gather(ev_v, [i1])
                vw1 = plsc.load_gather(vec_v, [i1])
                r0 = x0 - ew0
                r1 = x1 - ew1
                e0 = jnp.exp(tc0 * (dmin0 - r0 * r0))
                e1 = jnp.exp(tc1 * (dmin1 - r1 * r1))
                return (n0 + vw0 * e0, d0 + e0, n1 + vw1 * e1, d1 + e1)

            zero = jnp.zeros((L,), jnp.float32)
            n0, d0, n1, d1 = lax.fori_loop(
                0, W, wstep, (zero, zero, zero, zero), unroll=8)
            o_v[pl.ds(o0, L)] = n0 / d0
            o_v[pl.ds(o1, L)] = n1 / d1
            return 0

        lax.fori_loop(0, GROUPS // 2, window_pair, 0)
        pltpu.sync_copy(o_v, out_hbm.at[pl.ds(base, BPW)])

    return hwnet_sc


def kernel(inputs, evaluate_table, takecare_table, vector_table, idx_table):
    B = inputs.shape[0]
    T = evaluate_table.shape[0]
    E = (idx_table.shape[0] - 1) // 2
    D = vector_table.shape[1]
    assert D == 1
    fn = _build(B, T, E)
    out = fn(
        inputs.reshape(B),
        evaluate_table.reshape(T),
        takecare_table.reshape(T),
        vector_table.reshape(T),
    )
    return out.reshape(B, D)


# speedup vs baseline: 355.0003x; 1.0134x over previous
import functools

import jax
import jax.numpy as jnp
from jax import lax
from jax.experimental import pallas as pl
from jax.experimental.pallas import tpu as pltpu
from jax.experimental.pallas import tpu_sc as plsc

L = 16


@functools.lru_cache(maxsize=None)
def _build(B, T, E):
    W = 2 * E + 1
    info = plsc.get_sparse_core_info()
    NC, NS = info.num_cores, info.num_subcores
    NW = NC * NS
    BPW = B // NW
    GROUPS = BPW // L
    mesh = plsc.VectorSubcoreMesh(core_axis_name="c", subcore_axis_name="s")

    @functools.partial(
        pl.kernel,
        mesh=mesh,
        out_type=jax.ShapeDtypeStruct((B,), jnp.float32),
        compiler_params=pltpu.CompilerParams(needs_layout_passes=False),
        scratch_types=[
            pltpu.VMEM((T,), jnp.float32),
            pltpu.VMEM((T,), jnp.float32),
            pltpu.VMEM((BPW,), jnp.float32),
            pltpu.VMEM((BPW,), jnp.float32),
            pltpu.VMEM((BPW,), jnp.int32),
            pltpu.VMEM((BPW,), jnp.int32),
            pltpu.VMEM((BPW,), jnp.float32),
            pltpu.VMEM((BPW,), jnp.float32),
            pltpu.SemaphoreType.DMA,
            pltpu.SemaphoreType.DMA,
            pltpu.SemaphoreType.DMA,
        ],
    )
    def hwnet_sc(x_hbm, ev_hbm, tc_hbm, vec_hbm, out_hbm,
                 ev_v, vec_v, x_v, o_v, idx_v, st_v, dm_v, tcs_v,
                 sem_ev, sem_vec, sem_x):
        wid = lax.axis_index("s") * NC + lax.axis_index("c")
        base = wid * BPW
        h_ev = pltpu.async_copy(ev_hbm, ev_v, sem_ev)
        h_vec = pltpu.async_copy(vec_hbm, vec_v, sem_vec)
        h_x = pltpu.async_copy(x_hbm.at[pl.ds(base, BPW)], x_v, sem_x)
        h_ev.wait()
        h_x.wait()

        LOG = T.bit_length() - 1

        def lower_bound2(k0, k1):
            def body(i, carry):
                p0, p1, half = carry
                hm1 = half - 1
                v0 = plsc.load_gather(ev_v, [p0 + hm1])
                v1 = plsc.load_gather(ev_v, [p1 + hm1])
                p0 = jnp.where(v0 < k0, p0 + half, p0)
                p1 = jnp.where(v1 < k1, p1 + half, p1)
                return (p0, p1, half >> 1)

            z = jnp.zeros((L,), jnp.int32)
            h0 = jnp.full((L,), T // 2, jnp.int32)
            p0, p1, _ = lax.fori_loop(0, LOG, body, (z, z, h0))
            return p0, p1

        def nearest(x, pos):
            a = jnp.maximum(pos - 1, 0)
            ea = plsc.load_gather(ev_v, [a])
            eb = plsc.load_gather(ev_v, [pos])
            ra = x - ea
            rb = x - eb
            da = ra * ra
            db = rb * rb
            take_a = da <= db
            vstar = jnp.where(take_a, ea, eb)
            dmin = jnp.minimum(da, db)
            cand = jnp.where(take_a, a, pos)
            return vstar, dmin, cand

        def search_pair(jj, _):
            j0 = jj * 2
            x0 = x_v[pl.ds(j0 * L, L)]
            x1 = x_v[pl.ds(j0 * L + L, L)]
            pos0, pos1 = lower_bound2(x0, x1)
            v0, dmin0, c0 = nearest(x0, pos0)
            v1, dmin1, c1 = nearest(x1, pos1)
            pa0 = plsc.load_gather(ev_v, [jnp.maximum(c0 - 1, 0)])
            pa1 = plsc.load_gather(ev_v, [jnp.maximum(c1 - 1, 0)])
            dup0 = (pa0 == v0) & (c0 > 0)
            dup1 = (pa1 == v1) & (c1 > 0)
            any_dup = jnp.any(dup0) | jnp.any(dup1)

            def slow(_):
                f0, f1 = lower_bound2(v0, v1)
                return (jnp.where(dup0, f0, c0), jnp.where(dup1, f1, c1))

            idx0, idx1 = lax.cond(any_dup, slow, lambda _: (c0, c1), 0)

            idx_v[pl.ds(j0 * L, L)] = idx0
            idx_v[pl.ds(j0 * L + L, L)] = idx1
            st_v[pl.ds(j0 * L, L)] = jnp.clip(idx0, E, T - E - 1) - E
            st_v[pl.ds(j0 * L + L, L)] = jnp.clip(idx1, E, T - E - 1) - E
            dm_v[pl.ds(j0 * L, L)] = dmin0
            dm_v[pl.ds(j0 * L + L, L)] = dmin1
            return 0

        lax.fori_loop(0, GROUPS // 2, search_pair, 0)

        pltpu.async_copy(tc_hbm.at[idx_v], tcs_v, sem_ev).wait()
        h_vec.wait()

        def window_pair(jj, _):
            j0 = jj * 2
            o0 = j0 * L
            o1 = j0 * L + L
            x0 = x_v[pl.ds(o0, L)]
            x1 = x_v[pl.ds(o1, L)]
            s0 = st_v[pl.ds(o0, L)]
            s1 = st_v[pl.ds(o1, L)]
            dmin0 = dm_v[pl.ds(o0, L)]
            dmin1 = dm_v[pl.ds(o1, L)]
            tc0 = tcs_v[pl.ds(o0, L)]
            tc1 = tcs_v[pl.ds(o1, L)]

            def wstep(w, carry):
                n0, d0, n1, d1 = carry
                i0 = s0 + w
                i1 = s1 + w
                ew0 = plsc.load_gather(ev_v, [i0])
                vw0 = plsc.load_gather(vec_v, [i0])
                ew1 = plsc.load_gather(ev_v, [i1])
                vw1 = plsc.load_gather(vec_v, [i1])
                r0 = x0 - ew0
                r1 = x1 - ew1
                e0 = jnp.exp(tc0 * (dmin0 - r0 * r0))
                e1 = jnp.exp(tc1 * (dmin1 - r1 * r1))
                return (n0 + vw0 * e0, d0 + e0, n1 + vw1 * e1, d1 + e1)

            zero = jnp.zeros((L,), jnp.float32)
            n0, d0, n1, d1 = lax.fori_loop(
                0, W, wstep, (zero, zero, zero, zero), unroll=2)
            o_v[pl.ds(o0, L)] = n0 / d0
            o_v[pl.ds(o1, L)] = n1 / d1
            return 0

        lax.fori_loop(0, GROUPS // 2, window_pair, 0)
        pltpu.sync_copy(o_v, out_hbm.at[pl.ds(base, BPW)])

    return hwnet_sc


def kernel(inputs, evaluate_table, takecare_table, vector_table, idx_table):
    B = inputs.shape[0]
    T = evaluate_table.shape[0]
    E = (idx_table.shape[0] - 1) // 2
    D = vector_table.shape[1]
    assert D == 1
    fn = _build(B, T, E)
    out = fn(
        inputs.reshape(B),
        evaluate_table.reshape(T),
        takecare_table.reshape(T),
        vector_table.reshape(T),
    )
    return out.reshape(B, D)


# speedup vs baseline: 357.7199x; 1.0077x over previous
import functools

import jax
import jax.numpy as jnp
from jax import lax
from jax.experimental import pallas as pl
from jax.experimental.pallas import tpu as pltpu
from jax.experimental.pallas import tpu_sc as plsc

L = 16


@functools.lru_cache(maxsize=None)
def _build(B, T, E):
    W = 2 * E + 1
    info = plsc.get_sparse_core_info()
    NC, NS = info.num_cores, info.num_subcores
    NW = NC * NS
    BPW = B // NW
    GROUPS = BPW // L
    mesh = plsc.VectorSubcoreMesh(core_axis_name="c", subcore_axis_name="s")

    @functools.partial(
        pl.kernel,
        mesh=mesh,
        out_type=jax.ShapeDtypeStruct((B,), jnp.float32),
        compiler_params=pltpu.CompilerParams(needs_layout_passes=False),
        scratch_types=[
            pltpu.VMEM((T,), jnp.float32),
            pltpu.VMEM((T,), jnp.float32),
            pltpu.VMEM((BPW,), jnp.float32),
            pltpu.VMEM((BPW,), jnp.float32),
            pltpu.VMEM((BPW,), jnp.int32),
            pltpu.VMEM((BPW,), jnp.int32),
            pltpu.VMEM((BPW,), jnp.float32),
            pltpu.VMEM((BPW,), jnp.float32),
            pltpu.SemaphoreType.DMA,
            pltpu.SemaphoreType.DMA,
            pltpu.SemaphoreType.DMA,
        ],
    )
    def hwnet_sc(x_hbm, ev_hbm, tc_hbm, vec_hbm, out_hbm,
                 ev_v, vec_v, x_v, o_v, idx_v, st_v, dm_v, tcs_v,
                 sem_ev, sem_vec, sem_x):
        wid = lax.axis_index("s") * NC + lax.axis_index("c")
        base = wid * BPW
        h_ev = pltpu.async_copy(ev_hbm, ev_v, sem_ev)
        h_vec = pltpu.async_copy(vec_hbm, vec_v, sem_vec)
        h_x = pltpu.async_copy(x_hbm.at[pl.ds(base, BPW)], x_v, sem_x)
        h_ev.wait()
        h_x.wait()

        LOG = T.bit_length() - 1

        def lower_bound2(k0, k1):
            def body(i, carry):
                p0, p1, half = carry
                hm1 = half - 1
                v0 = plsc.load_gather(ev_v, [p0 + hm1])
                v1 = plsc.load_gather(ev_v, [p1 + hm1])
                p0 = jnp.where(v0 < k0, p0 + half, p0)
                p1 = jnp.where(v1 < k1, p1 + half, p1)
                return (p0, p1, half >> 1)

            z = jnp.zeros((L,), jnp.int32)
            h0 = jnp.full((L,), T // 2, jnp.int32)
            p0, p1, _ = lax.fori_loop(0, LOG, body, (z, z, h0))
            return p0, p1

        def nearest(x, pos):
            a = jnp.maximum(pos - 1, 0)
            ea = plsc.load_gather(ev_v, [a])
            eb = plsc.load_gather(ev_v, [pos])
            ra = x - ea
            rb = x - eb
            da = ra * ra
            db = rb * rb
            take_a = da <= db
            vstar = jnp.where(take_a, ea, eb)
            dmin = jnp.minimum(da, db)
            cand = jnp.where(take_a, a, pos)
            return vstar, dmin, cand

        def search_pair(jj, _):
            j0 = jj * 2
            x0 = x_v[pl.ds(j0 * L, L)]
            x1 = x_v[pl.ds(j0 * L + L, L)]
            pos0, pos1 = lower_bound2(x0, x1)
            v0, dmin0, c0 = nearest(x0, pos0)
            v1, dmin1, c1 = nearest(x1, pos1)
            pa0 = plsc.load_gather(ev_v, [jnp.maximum(c0 - 1, 0)])
            pa1 = plsc.load_gather(ev_v, [jnp.maximum(c1 - 1, 0)])
            dup0 = (pa0 == v0) & (c0 > 0)
            dup1 = (pa1 == v1) & (c1 > 0)
            any_dup = jnp.any(dup0) | jnp.any(dup1)

            def slow(_):
                f0, f1 = lower_bound2(v0, v1)
                return (jnp.where(dup0, f0, c0), jnp.where(dup1, f1, c1))

            idx0, idx1 = lax.cond(any_dup, slow, lambda _: (c0, c1), 0)

            idx_v[pl.ds(j0 * L, L)] = idx0
            idx_v[pl.ds(j0 * L + L, L)] = idx1
            st_v[pl.ds(j0 * L, L)] = jnp.clip(idx0, E, T - E - 1) - E
            st_v[pl.ds(j0 * L + L, L)] = jnp.clip(idx1, E, T - E - 1) - E
            dm_v[pl.ds(j0 * L, L)] = dmin0
            dm_v[pl.ds(j0 * L + L, L)] = dmin1
            return 0

        lax.fori_loop(0, GROUPS // 2, search_pair, 0)

        pltpu.async_copy(tc_hbm.at[idx_v], tcs_v, sem_ev).wait()
        h_vec.wait()

        def window_pair(jj, _):
            j0 = jj * 2
            o0 = j0 * L
            o1 = j0 * L + L
            x0 = x_v[pl.ds(o0, L)]
            x1 = x_v[pl.ds(o1, L)]
            s0 = st_v[pl.ds(o0, L)]
            s1 = st_v[pl.ds(o1, L)]
            dmin0 = dm_v[pl.ds(o0, L)]
            dmin1 = dm_v[pl.ds(o1, L)]
            tc0 = tcs_v[pl.ds(o0, L)]
            tc1 = tcs_v[pl.ds(o1, L)]

            zero = jnp.zeros((L,), jnp.float32)

            @plsc.parallel_loop(0, W, 1, unroll=4,
                                carry=(zero, zero, zero, zero))
            def wresult(w, carry):
                n0, d0, n1, d1 = carry
                i0 = s0 + w
                i1 = s1 + w
                ew0 = plsc.load_gather(ev_v, [i0])
                vw0 = plsc.load_gather(vec_v, [i0])
                ew1 = plsc.load_gather(ev_v, [i1])
                vw1 = plsc.load_gather(vec_v, [i1])
                r0 = x0 - ew0
                r1 = x1 - ew1
                e0 = jnp.exp(tc0 * (dmin0 - r0 * r0))
                e1 = jnp.exp(tc1 * (dmin1 - r1 * r1))
                return (n0 + vw0 * e0, d0 + e0, n1 + vw1 * e1, d1 + e1)

            n0, d0, n1, d1 = wresult
            o_v[pl.ds(o0, L)] = n0 / d0
            o_v[pl.ds(o1, L)] = n1 / d1
            return 0

        lax.fori_loop(0, GROUPS // 2, window_pair, 0)
        pltpu.sync_copy(o_v, out_hbm.at[pl.ds(base, BPW)])

    return hwnet_sc


def kernel(inputs, evaluate_table, takecare_table, vector_table, idx_table):
    B = inputs.shape[0]
    T = evaluate_table.shape[0]
    E = (idx_table.shape[0] - 1) // 2
    D = vector_table.shape[1]
    assert D == 1
    fn = _build(B, T, E)
    out = fn(
        inputs.reshape(B),
        evaluate_table.reshape(T),
        takecare_table.reshape(T),
        vector_table.reshape(T),
    )
    return out.reshape(B, D)


# speedup vs baseline: 366.8588x; 1.0255x over previous
import functools

import jax
import jax.numpy as jnp
from jax import lax
from jax.experimental import pallas as pl
from jax.experimental.pallas import tpu as pltpu
from jax.experimental.pallas import tpu_sc as plsc

L = 16


@functools.lru_cache(maxsize=None)
def _build(B, T, E):
    W = 2 * E + 1
    info = plsc.get_sparse_core_info()
    NC, NS = info.num_cores, info.num_subcores
    NW = NC * NS
    BPW = B // NW
    GROUPS = BPW // L
    mesh = plsc.VectorSubcoreMesh(core_axis_name="c", subcore_axis_name="s")

    @functools.partial(
        pl.kernel,
        mesh=mesh,
        out_type=jax.ShapeDtypeStruct((B,), jnp.float32),
        compiler_params=pltpu.CompilerParams(needs_layout_passes=False),
        scratch_types=[
            pltpu.VMEM((2 * T,), jnp.float32),
            pltpu.VMEM((2 * BPW,), jnp.float32),
            pltpu.VMEM((2 * BPW,), jnp.int32),
            pltpu.VMEM((2 * BPW,), jnp.float32),
            pltpu.SemaphoreType.DMA,
            pltpu.SemaphoreType.DMA,
        ],
    )
    def hwnet_sc(tab_hbm, x_hbm, out_hbm, evvec, xo, ints, flts,
                 sem_t, sem_x):
        wid = lax.axis_index("s") * NC + lax.axis_index("c")
        base = wid * BPW
        h_t = pltpu.async_copy(tab_hbm.at[pl.ds(0, 2 * T)], evvec, sem_t)
        h_x = pltpu.async_copy(x_hbm.at[pl.ds(base, BPW)],
                               xo.at[pl.ds(0, BPW)], sem_x)
        h_x.wait()
        h_t.wait()

        LOG = T.bit_length() - 1

        def lower_bound2(k0, k1):
            def body(i, carry):
                p0, p1, half = carry
                hm1 = half - 1
                v0 = plsc.load_gather(evvec, [p0 + hm1])
                v1 = plsc.load_gather(evvec, [p1 + hm1])
                p0 = jnp.where(v0 < k0, p0 + half, p0)
                p1 = jnp.where(v1 < k1, p1 + half, p1)
                return (p0, p1, half >> 1)

            z = jnp.zeros((L,), jnp.int32)
            h0 = jnp.full((L,), T // 2, jnp.int32)
            p0, p1, _ = lax.fori_loop(0, LOG, body, (z, z, h0))
            return p0, p1

        def nearest(x, pos):
            a = jnp.maximum(pos - 1, 0)
            ea = plsc.load_gather(evvec, [a])
            eb = plsc.load_gather(evvec, [pos])
            ra = x - ea
            rb = x - eb
            da = ra * ra
            db = rb * rb
            take_a = da <= db
            vstar = jnp.where(take_a, ea, eb)
            dmin = jnp.minimum(da, db)
            cand = jnp.where(take_a, a, pos)
            return vstar, dmin, cand

        def search_pair(jj, _):
            j0 = jj * 2
            x0 = xo[pl.ds(j0 * L, L)]
            x1 = xo[pl.ds(j0 * L + L, L)]
            pos0, pos1 = lower_bound2(x0, x1)
            v0, dmin0, c0 = nearest(x0, pos0)
            v1, dmin1, c1 = nearest(x1, pos1)
            pa0 = plsc.load_gather(evvec, [jnp.maximum(c0 - 1, 0)])
            pa1 = plsc.load_gather(evvec, [jnp.maximum(c1 - 1, 0)])
            dup0 = (pa0 == v0) & (c0 > 0)
            dup1 = (pa1 == v1) & (c1 > 0)
            any_dup = jnp.any(dup0) | jnp.any(dup1)

            def slow(_):
                f0, f1 = lower_bound2(v0, v1)
                return (jnp.where(dup0, f0, c0), jnp.where(dup1, f1, c1))

            idx0, idx1 = lax.cond(any_dup, slow, lambda _: (c0, c1), 0)

            ints[pl.ds(j0 * L, L)] = idx0 + (2 * T)
            ints[pl.ds(j0 * L + L, L)] = idx1 + (2 * T)
            ints[pl.ds(BPW + j0 * L, L)] = jnp.clip(idx0, E, T - E - 1) - E
            ints[pl.ds(BPW + j0 * L + L, L)] = jnp.clip(idx1, E, T - E - 1) - E
            flts[pl.ds(j0 * L, L)] = dmin0
            flts[pl.ds(j0 * L + L, L)] = dmin1
            return 0

        lax.fori_loop(0, GROUPS // 2, search_pair, 0)

        pltpu.async_copy(tab_hbm.at[ints.at[pl.ds(0, BPW)]],
                         flts.at[pl.ds(BPW, BPW)], sem_x).wait()

        def window_pair(jj, _):
            j0 = jj * 2
            o0 = j0 * L
            o1 = j0 * L + L
            x0 = xo[pl.ds(o0, L)]
            x1 = xo[pl.ds(o1, L)]
            s0 = ints[pl.ds(BPW + o0, L)]
            s1 = ints[pl.ds(BPW + o1, L)]
            dmin0 = flts[pl.ds(o0, L)]
            dmin1 = flts[pl.ds(o1, L)]
            tc0 = flts[pl.ds(BPW + o0, L)]
            tc1 = flts[pl.ds(BPW + o1, L)]

            zero = jnp.zeros((L,), jnp.float32)

            @plsc.parallel_loop(0, W, 1, unroll=4,
                                carry=(zero, zero, zero, zero))
            def wresult(w, carry):
                n0, d0, n1, d1 = carry
                i0 = s0 + w
                i1 = s1 + w
                ew0 = plsc.load_gather(evvec, [i0])
                vw0 = plsc.load_gather(evvec, [i0 + T])
                ew1 = plsc.load_gather(evvec, [i1])
                vw1 = plsc.load_gather(evvec, [i1 + T])
                r0 = x0 - ew0
                r1 = x1 - ew1
                e0 = jnp.exp(tc0 * (dmin0 - r0 * r0))
                e1 = jnp.exp(tc1 * (dmin1 - r1 * r1))
                return (n0 + vw0 * e0, d0 + e0, n1 + vw1 * e1, d1 + e1)

            n0, d0, n1, d1 = wresult
            xo[pl.ds(BPW + o0, L)] = n0 / d0
            xo[pl.ds(BPW + o1, L)] = n1 / d1
            return 0

        lax.fori_loop(0, GROUPS // 2, window_pair, 0)
        pltpu.sync_copy(xo.at[pl.ds(BPW, BPW)],
                        out_hbm.at[pl.ds(base, BPW)])

    return hwnet_sc


def kernel(inputs, evaluate_table, takecare_table, vector_table, idx_table):
    B = inputs.shape[0]
    T = evaluate_table.shape[0]
    E = (idx_table.shape[0] - 1) // 2
    D = vector_table.shape[1]
    assert D == 1
    tables = jnp.concatenate([
        evaluate_table.reshape(T),
        vector_table.reshape(T),
        takecare_table.reshape(T),
    ])
    fn = _build(B, T, E)
    out = fn(tables, inputs.reshape(B))
    return out.reshape(B, D)
